# baseline (speedup 1.0000x reference)
import jax
import jax.numpy as jnp
from jax import lax
from jax.experimental import pallas as pl
from jax.experimental.pallas import tpu as pltpu
from jax.experimental.pallas import tpu_sc as plsc

NLEV = 16
NFEAT = 2
HSIZE = 524288
BATCH = 524288
RES = [16, 22, 30, 42, 58, 80, 110, 152, 210, 290, 400, 552, 762, 1052, 1452, 2048]
P1 = -1640531535
P2 = 805459861
MASK = HSIZE - 1

NC, NS = 2, 16
NW = NC * NS
NPW = BATCH // NW
EPL = NPW * NFEAT

C1 = 4096
N1 = NPW // C1
EC1 = C1 * NFEAT
HS1 = C1 // 16
GROWS = 4096
NG1 = EC1 // GROWS

C2 = 256
N2 = NPW // C2
SEG = C2 * NFEAT
HALF = NLEV * SEG


def _sc_body(coords_hbm, tflat_hbm, out_hbm, tmp_hbm, cbuf, ibuf, vbuf, spt, sem):
    cid = lax.axis_index("c")
    sid = lax.axis_index("s")
    wid = sid * jnp.int32(NC) + cid
    wbase = wid * jnp.int32(NPW)
    lanes = lax.iota(jnp.int32, 16)
    lanes2 = lanes * jnp.int32(2)
    t_lvl = lax.shift_right_logical(lanes, jnp.int32(1)) * jnp.int32(SEG)
    t_feat = lanes & jnp.int32(1)

    for lvl in range(NLEV):
        plsc.subcore_barrier()

        @pl.when(sid == jnp.int32(0))
        def _():
            pltpu.sync_copy(
                tflat_hbm.at[pl.ds(lvl * HSIZE * NFEAT, HSIZE * NFEAT)], spt)

        plsc.subcore_barrier()

        r = jnp.float32(RES[lvl])

        def chunk1(k, carry, r=r, lvl=lvl):
            base = wbase + k * jnp.int32(C1)
            pltpu.sync_copy(coords_hbm.at[:, pl.ds(base, C1)], cbuf)

            def hash_step(s, carry2):
                col = s * jnp.int32(16)
                x = cbuf[0, pl.ds(col, 16)]
                y = cbuf[1, pl.ds(col, 16)]
                z = cbuf[2, pl.ds(col, 16)]
                gx = (x * r).astype(jnp.int32)
                gy = (y * r).astype(jnp.int32)
                gz = (z * r).astype(jnp.int32)
                h = gx ^ (gy * jnp.int32(P1)) ^ (gz * jnp.int32(P2))
                e0 = (h & jnp.int32(MASK)) * jnp.int32(2)
                pb = lanes2 + s * jnp.int32(32)
                plsc.store_scatter(ibuf, [pb], e0)
                plsc.store_scatter(ibuf, [pb + jnp.int32(1)],
                                   e0 + jnp.int32(1))
                return carry2

            lax.fori_loop(jnp.int32(0), jnp.int32(HS1), hash_step,
                          jnp.int32(0))

            for j in range(NG1):
                pltpu.async_copy(
                    spt.at[ibuf.at[pl.ds(j * GROWS, GROWS)]],
                    vbuf.at[pl.ds(j * GROWS, GROWS)],
                    sem)
            for j in range(NG1):
                pltpu.make_async_copy(
                    spt.at[ibuf.at[pl.ds(j * GROWS, GROWS)]],
                    vbuf.at[pl.ds(j * GROWS, GROWS)],
                    sem).wait()

            tbase = ((wid * jnp.int32(NLEV) + jnp.int32(lvl))
                     * jnp.int32(EPL) + k * jnp.int32(EC1))
            pltpu.sync_copy(vbuf.at[pl.ds(0, EC1)],
                            tmp_hbm.at[pl.ds(tbase, EC1)])
            return carry

        lax.fori_loop(jnp.int32(0), jnp.int32(N1), chunk1, jnp.int32(0))

    def chunk2(k, carry):
        for lvl in range(NLEV):
            tbase = ((wid * jnp.int32(NLEV) + jnp.int32(lvl))
                     * jnp.int32(EPL) + k * jnp.int32(SEG))
            pltpu.sync_copy(tmp_hbm.at[pl.ds(tbase, SEG)],
                            vbuf.at[pl.ds(lvl * SEG, SEG)])

        def asm(c, carry2):
            src0 = t_lvl + t_feat + c * jnp.int32(2)
            lo = plsc.load_gather(vbuf, [src0])
            hi = plsc.load_gather(vbuf, [src0 + jnp.int32(8 * SEG)])
            ob = jnp.int32(HALF) + c * jnp.int32(32)
            vbuf[pl.ds(ob, 16)] = lo
            vbuf[pl.ds(ob + jnp.int32(16), 16)] = hi
            return carry2

        lax.fori_loop(jnp.int32(0), jnp.int32(C2), asm, jnp.int32(0))

        obase = (wbase + k * jnp.int32(C2)) * jnp.int32(NLEV * NFEAT)
        pltpu.sync_copy(vbuf.at[pl.ds(HALF, C2 * NLEV * NFEAT)],
                        out_hbm.at[pl.ds(obase, C2 * NLEV * NFEAT)])
        return carry

    lax.fori_loop(jnp.int32(0), jnp.int32(N2), chunk2, jnp.int32(0))


def kernel(coords, tables):
    coords_t = coords.T.astype(jnp.float32)
    tflat = tables.reshape(NLEV * HSIZE * NFEAT)
    mesh = plsc.VectorSubcoreMesh(core_axis_name="c", subcore_axis_name="s")
    f = pl.kernel(
        _sc_body,
        mesh=mesh,
        compiler_params=pltpu.CompilerParams(
            use_tc_tiling_on_sc=False, needs_layout_passes=False),
        out_type=(
            jax.ShapeDtypeStruct((BATCH * NLEV * NFEAT,), jnp.float32),
            jax.ShapeDtypeStruct((BATCH * NLEV * NFEAT,), jnp.float32),
        ),
        scratch_types=[
            pltpu.VMEM((3, C1), jnp.float32),
            pltpu.VMEM((EC1,), jnp.int32),
            pltpu.VMEM((2 * HALF,), jnp.float32),
            pltpu.VMEM_SHARED((HSIZE * NFEAT,), jnp.float32),
            pltpu.SemaphoreType.DMA,
        ],
    )
    out, _ = f(coords_t, tflat)
    return out.reshape(BATCH, NLEV * NFEAT)
